# baseline (speedup 1.0000x reference)
import functools

import jax
import jax.numpy as jnp
from jax import lax
from jax.experimental import pallas as pl
from jax.experimental.pallas import tpu as pltpu
from jax.experimental.pallas import tpu_sc as plsc

N = 10000
E = 320000
D = 128

NC = 2
NS = 16
NW = NC * NS
CHUNK = 125
EPT = E // NW
ITERS = EPT // CHUNK
STAGES = 2
SITERS = ITERS // STAGES
PAIRS = SITERS // 2
RA = 632
RB = N - (NS - 1) * RA


def _combine_body(p_ref, o_ref):
    o_ref[...] = p_ref[:N] + p_ref[N:]


def _combine(p):
    return pl.pallas_call(
        _combine_body,
        out_shape=jax.ShapeDtypeStruct((N, D), jnp.float32),
    )(p)


def _combine_mlp_body(p_ref, w1_ref, b1_ref, w2_ref, b2_ref, o_ref):
    h = p_ref[:N] + p_ref[N:]
    h = lax.dot_general(h, w1_ref[...], (((1,), (1,)), ((), ())),
                        preferred_element_type=jnp.float32) + b1_ref[...]
    o_ref[...] = lax.dot_general(h, w2_ref[...], (((1,), (1,)), ((), ())),
                                 preferred_element_type=jnp.float32) + b2_ref[...]


def _combine_mlp(p, W1, b1row, W2, b2row):
    return pl.pallas_call(
        _combine_mlp_body,
        out_shape=jax.ShapeDtypeStruct((N, D), jnp.float32),
    )(p, W1, b1row, W2, b2row)


_MESH = plsc.VectorSubcoreMesh(core_axis_name="c", subcore_axis_name="s")


@functools.partial(
    pl.kernel,
    mesh=_MESH,
    out_type=jax.ShapeDtypeStruct((NC * N, D), jnp.float32),
    scratch_types=[
        pltpu.VMEM((SITERS, CHUNK), jnp.int32),
        pltpu.VMEM((SITERS, CHUNK), jnp.int32),
        pltpu.VMEM((CHUNK, D), jnp.float32),
        pltpu.VMEM((CHUNK, D), jnp.float32),
        pltpu.VMEM_SHARED((N, D), jnp.float32),
        pltpu.SemaphoreType.DMA,
        pltpu.SemaphoreType.DMA,
        pltpu.SemaphoreType.DMA,
        pltpu.SemaphoreType.DMA,
    ],
)
def _spmm(src_hbm, dst_hbm, g_hbm, z_hbm, out_hbm,
          sidx, didx, rows0, rows1, acc, gsem0, gsem1, ssem0, ssem1):
    cid = lax.axis_index("c")
    sid = lax.axis_index("s")
    wid = cid * NS + sid

    @pl.when(sid < NS - 1)
    def _():
        r0 = pl.multiple_of(sid * RA, 8)
        pltpu.sync_copy(z_hbm, acc.at[pl.ds(r0, RA)])

    @pl.when(sid == NS - 1)
    def _():
        pltpu.sync_copy(z_hbm.at[pl.ds(0, RB)], acc.at[pl.ds(N - RB, RB)])

    plsc.subcore_barrier()

    for s in range(STAGES):
        pltpu.sync_copy(src_hbm.at[wid, pl.ds(s * SITERS, SITERS)], sidx)
        pltpu.sync_copy(dst_hbm.at[wid, pl.ds(s * SITERS, SITERS)], didx)
        pltpu.async_copy(g_hbm.at[sidx.at[0]], rows0, gsem0)
        pltpu.async_copy(g_hbm.at[sidx.at[1]], rows1, gsem1)

        def body(p, carry):
            i0 = 2 * p
            pltpu.make_async_copy(g_hbm.at[sidx.at[i0]], rows0, gsem0).wait()
            pltpu.async_copy(rows0, acc.at[didx.at[i0]], ssem0, add=True)
            pltpu.make_async_copy(g_hbm.at[sidx.at[i0 + 1]], rows1, gsem1).wait()
            pltpu.async_copy(rows1, acc.at[didx.at[i0 + 1]], ssem1, add=True)

            pltpu.make_async_copy(rows0, acc.at[didx.at[i0]], ssem0).wait()

            @pl.when(p < PAIRS - 1)
            def _():
                pltpu.async_copy(g_hbm.at[sidx.at[i0 + 2]], rows0, gsem0)

            pltpu.make_async_copy(rows1, acc.at[didx.at[i0 + 1]], ssem1).wait()

            @pl.when(p < PAIRS - 1)
            def _():
                pltpu.async_copy(g_hbm.at[sidx.at[i0 + 3]], rows1, gsem1)

            return carry

        lax.fori_loop(0, PAIRS, body, 0)
    plsc.subcore_barrier()

    @pl.when(sid < NS - 1)
    def _():
        r0 = pl.multiple_of(sid * RA, 8)
        o0 = pl.multiple_of(cid * N + sid * RA, 8)
        pltpu.sync_copy(acc.at[pl.ds(r0, RA)], out_hbm.at[pl.ds(o0, RA)])

    @pl.when(sid == NS - 1)
    def _():
        o0 = pl.multiple_of(cid * N + (N - RB), 8)
        pltpu.sync_copy(acc.at[pl.ds(N - RB, RB)], out_hbm.at[pl.ds(o0, RB)])


def kernel(feat, edge_index, W1, b1, W2, b2):
    src = edge_index[0].reshape(NW, ITERS, CHUNK)
    dst = edge_index[1].reshape(NW, ITERS, CHUNK)
    zeros_slab = jnp.zeros((RA, D), jnp.float32)
    b1row = b1.reshape(1, D)
    b2row = b2.reshape(1, D)

    p = _spmm(src, dst, feat, zeros_slab)
    h1 = _combine(p)
    p2 = _spmm(src, dst, h1, zeros_slab)
    return _combine_mlp(p2, W1, b1row, W2, b2row)


# speedup vs baseline: 1.2203x; 1.2203x over previous
import functools

import jax
import jax.numpy as jnp
from jax import lax
from jax.experimental import pallas as pl
from jax.experimental.pallas import tpu as pltpu
from jax.experimental.pallas import tpu_sc as plsc

N = 10000
E = 320000
D = 128

NC = 2
NS = 16
NW = NC * NS
CHUNK = 125
EPT = E // NW
ITERS = EPT // CHUNK
STAGES = 5
SITERS = ITERS // STAGES
PAIRS = SITERS // 2
RA = 632
RB = N - (NS - 1) * RA


def _combine_body(p_ref, o_ref):
    o_ref[...] = p_ref[:N] + p_ref[N:]


def _combine(p):
    return pl.pallas_call(
        _combine_body,
        out_shape=jax.ShapeDtypeStruct((N, D), jnp.float32),
    )(p)


def _combine_mlp_body(p_ref, w1_ref, b1_ref, w2_ref, b2_ref, o_ref):
    h = p_ref[:N] + p_ref[N:]
    h = lax.dot_general(h, w1_ref[...], (((1,), (1,)), ((), ())),
                        preferred_element_type=jnp.float32) + b1_ref[...]
    o_ref[...] = lax.dot_general(h, w2_ref[...], (((1,), (1,)), ((), ())),
                                 preferred_element_type=jnp.float32) + b2_ref[...]


def _combine_mlp(p, W1, b1row, W2, b2row):
    return pl.pallas_call(
        _combine_mlp_body,
        out_shape=jax.ShapeDtypeStruct((N, D), jnp.float32),
    )(p, W1, b1row, W2, b2row)


_MESH = plsc.VectorSubcoreMesh(core_axis_name="c", subcore_axis_name="s")


@functools.partial(
    pl.kernel,
    mesh=_MESH,
    out_type=jax.ShapeDtypeStruct((NC * N, D), jnp.float32),
    scratch_types=[
        pltpu.VMEM((SITERS, CHUNK), jnp.int32),
        pltpu.VMEM((SITERS, CHUNK), jnp.int32),
        pltpu.VMEM((SITERS, CHUNK), jnp.int32),
        pltpu.VMEM((SITERS, CHUNK), jnp.int32),
        pltpu.VMEM((CHUNK, D), jnp.float32),
        pltpu.VMEM((CHUNK, D), jnp.float32),
        pltpu.VMEM_SHARED((N, D), jnp.float32),
        pltpu.SemaphoreType.DMA,
        pltpu.SemaphoreType.DMA,
        pltpu.SemaphoreType.DMA,
        pltpu.SemaphoreType.DMA,
    ],
)
def _spmm(src_hbm, dst_hbm, g_hbm, z_hbm, out_hbm,
          sidxA, didxA, sidxB, didxB, rows0, rows1, acc,
          sem0, sem1, isem, zsem):
    cid = lax.axis_index("c")
    sid = lax.axis_index("s")
    wid = cid * NS + sid

    @pl.when(sid < NS - 1)
    def _():
        r0 = pl.multiple_of(sid * RA, 8)
        pltpu.async_copy(z_hbm, acc.at[pl.ds(r0, RA)], zsem)

    @pl.when(sid == NS - 1)
    def _():
        pltpu.async_copy(z_hbm.at[pl.ds(0, RB)], acc.at[pl.ds(N - RB, RB)], zsem)

    pltpu.sync_copy(src_hbm.at[wid, pl.ds(0, SITERS)], sidxA)
    pltpu.sync_copy(dst_hbm.at[wid, pl.ds(0, SITERS)], didxA)
    pltpu.async_copy(src_hbm.at[wid, pl.ds(SITERS, SITERS)], sidxB, isem)
    pltpu.async_copy(dst_hbm.at[wid, pl.ds(SITERS, SITERS)], didxB, isem)

    @pl.when(sid < NS - 1)
    def _():
        r0 = pl.multiple_of(sid * RA, 8)
        pltpu.make_async_copy(z_hbm, acc.at[pl.ds(r0, RA)], zsem).wait()

    @pl.when(sid == NS - 1)
    def _():
        pltpu.make_async_copy(z_hbm.at[pl.ds(0, RB)],
                              acc.at[pl.ds(N - RB, RB)], zsem).wait()

    plsc.subcore_barrier()

    for s in range(STAGES):
        sidx, didx = (sidxA, didxA) if s % 2 == 0 else (sidxB, didxB)
        nsidx, ndidx = (sidxB, didxB) if s % 2 == 0 else (sidxA, didxA)
        if s > 0:
            pltpu.make_async_copy(src_hbm.at[wid, pl.ds(s * SITERS, SITERS)],
                                  sidx, isem).wait()
            pltpu.make_async_copy(dst_hbm.at[wid, pl.ds(s * SITERS, SITERS)],
                                  didx, isem).wait()
        pltpu.async_copy(g_hbm.at[sidx.at[0]], rows0, sem0)
        if 0 < s < STAGES - 1:
            pltpu.async_copy(src_hbm.at[wid, pl.ds((s + 1) * SITERS, SITERS)],
                             nsidx, isem)
            pltpu.async_copy(dst_hbm.at[wid, pl.ds((s + 1) * SITERS, SITERS)],
                             ndidx, isem)

        def body(p, carry):
            i0 = 2 * p
            pltpu.async_copy(g_hbm.at[sidx.at[i0 + 1]], rows1, sem1)
            pltpu.make_async_copy(g_hbm.at[sidx.at[i0]], rows0, sem0).wait()
            pltpu.sync_copy(rows0, acc.at[didx.at[i0]], add=True)

            @pl.when(p < PAIRS - 1)
            def _():
                pltpu.async_copy(g_hbm.at[sidx.at[i0 + 2]], rows0, sem0)

            pltpu.make_async_copy(g_hbm.at[sidx.at[i0 + 1]], rows1, sem1).wait()
            pltpu.sync_copy(rows1, acc.at[didx.at[i0 + 1]], add=True)
            return carry

        lax.fori_loop(0, PAIRS, body, 0)
    plsc.subcore_barrier()

    @pl.when(sid < NS - 1)
    def _():
        r0 = pl.multiple_of(sid * RA, 8)
        o0 = pl.multiple_of(cid * N + sid * RA, 8)
        pltpu.sync_copy(acc.at[pl.ds(r0, RA)], out_hbm.at[pl.ds(o0, RA)])

    @pl.when(sid == NS - 1)
    def _():
        o0 = pl.multiple_of(cid * N + (N - RB), 8)
        pltpu.sync_copy(acc.at[pl.ds(N - RB, RB)], out_hbm.at[pl.ds(o0, RB)])


def kernel(feat, edge_index, W1, b1, W2, b2):
    src = edge_index[0].reshape(NW, ITERS, CHUNK)
    dst = edge_index[1].reshape(NW, ITERS, CHUNK)
    zeros_slab = jnp.zeros((RA, D), jnp.float32)
    b1row = b1.reshape(1, D)
    b2row = b2.reshape(1, D)

    p = _spmm(src, dst, feat, zeros_slab)
    h1 = _combine(p)
    p2 = _spmm(src, dst, h1, zeros_slab)
    return _combine_mlp(p2, W1, b1row, W2, b2row)


# speedup vs baseline: 1.2578x; 1.0307x over previous
import functools

import jax
import jax.numpy as jnp
from jax import lax
from jax.experimental import pallas as pl
from jax.experimental.pallas import tpu as pltpu
from jax.experimental.pallas import tpu_sc as plsc

N = 10000
E = 320000
D = 128

NC = 2
NS = 16
NW = NC * NS
CHUNK = 125
EPT = E // NW
ITERS = EPT // CHUNK
STAGES = 2
SITERS = ITERS // STAGES
PAIRS = SITERS // 2
RA = 632
RB = N - (NS - 1) * RA


def _combine_body(p_ref, o_ref):
    o_ref[...] = p_ref[:N] + p_ref[N:]


def _combine(p):
    return pl.pallas_call(
        _combine_body,
        out_shape=jax.ShapeDtypeStruct((N, D), jnp.float32),
    )(p)


def _combine_mlp_body(p_ref, w1_ref, b1_ref, w2_ref, b2_ref, o_ref):
    h = p_ref[:N] + p_ref[N:]
    h = lax.dot_general(h, w1_ref[...], (((1,), (1,)), ((), ())),
                        preferred_element_type=jnp.float32) + b1_ref[...]
    o_ref[...] = lax.dot_general(h, w2_ref[...], (((1,), (1,)), ((), ())),
                                 preferred_element_type=jnp.float32) + b2_ref[...]


def _combine_mlp(p, W1, b1row, W2, b2row):
    return pl.pallas_call(
        _combine_mlp_body,
        out_shape=jax.ShapeDtypeStruct((N, D), jnp.float32),
    )(p, W1, b1row, W2, b2row)


_MESH = plsc.VectorSubcoreMesh(core_axis_name="c", subcore_axis_name="s")


@functools.partial(
    pl.kernel,
    mesh=_MESH,
    out_type=jax.ShapeDtypeStruct((NC * N, D), jnp.float32),
    scratch_types=[
        pltpu.VMEM((SITERS, CHUNK), jnp.int32),
        pltpu.VMEM((SITERS, CHUNK), jnp.int32),
        pltpu.VMEM((CHUNK, D), jnp.float32),
        pltpu.VMEM((CHUNK, D), jnp.float32),
        pltpu.VMEM_SHARED((N, D), jnp.float32),
        pltpu.SemaphoreType.DMA,
        pltpu.SemaphoreType.DMA,
        pltpu.SemaphoreType.DMA,
    ],
)
def _spmm(src_hbm, dst_hbm, g_hbm, z_hbm, out_hbm,
          sidx, didx, rows0, rows1, acc, sem0, sem1, zsem):
    cid = lax.axis_index("c")
    sid = lax.axis_index("s")
    wid = cid * NS + sid

    @pl.when(sid < NS - 1)
    def _():
        r0 = pl.multiple_of(sid * RA, 8)
        pltpu.async_copy(z_hbm, acc.at[pl.ds(r0, RA)], zsem)

    @pl.when(sid == NS - 1)
    def _():
        pltpu.async_copy(z_hbm.at[pl.ds(0, RB)], acc.at[pl.ds(N - RB, RB)], zsem)

    @pl.when(sid < NS - 1)
    def _():
        r0 = pl.multiple_of(sid * RA, 8)
        pltpu.make_async_copy(z_hbm, acc.at[pl.ds(r0, RA)], zsem).wait()

    @pl.when(sid == NS - 1)
    def _():
        pltpu.make_async_copy(z_hbm.at[pl.ds(0, RB)],
                              acc.at[pl.ds(N - RB, RB)], zsem).wait()

    plsc.subcore_barrier()

    for s in range(STAGES):
        pltpu.sync_copy(src_hbm.at[wid, pl.ds(s * SITERS, SITERS)], sidx)
        pltpu.sync_copy(dst_hbm.at[wid, pl.ds(s * SITERS, SITERS)], didx)
        pltpu.async_copy(g_hbm.at[sidx.at[0]], rows0, sem0)

        def body(p, carry):
            i0 = 2 * p
            pltpu.async_copy(g_hbm.at[sidx.at[i0 + 1]], rows1, sem1)
            pltpu.make_async_copy(g_hbm.at[sidx.at[i0]], rows0, sem0).wait()
            pltpu.sync_copy(rows0, acc.at[didx.at[i0]], add=True)

            @pl.when(p < PAIRS - 1)
            def _():
                pltpu.async_copy(g_hbm.at[sidx.at[i0 + 2]], rows0, sem0)

            pltpu.make_async_copy(g_hbm.at[sidx.at[i0 + 1]], rows1, sem1).wait()
            pltpu.sync_copy(rows1, acc.at[didx.at[i0 + 1]], add=True)
            return carry

        lax.fori_loop(0, PAIRS, body, 0)
    plsc.subcore_barrier()

    @pl.when(sid < NS - 1)
    def _():
        r0 = pl.multiple_of(sid * RA, 8)
        o0 = pl.multiple_of(cid * N + sid * RA, 8)
        pltpu.sync_copy(acc.at[pl.ds(r0, RA)], out_hbm.at[pl.ds(o0, RA)])

    @pl.when(sid == NS - 1)
    def _():
        o0 = pl.multiple_of(cid * N + (N - RB), 8)
        pltpu.sync_copy(acc.at[pl.ds(N - RB, RB)], out_hbm.at[pl.ds(o0, RB)])


def kernel(feat, edge_index, W1, b1, W2, b2):
    src = edge_index[0].reshape(NW, ITERS, CHUNK)
    dst = edge_index[1].reshape(NW, ITERS, CHUNK)
    zeros_slab = jnp.zeros((RA, D), jnp.float32)
    b1row = b1.reshape(1, D)
    b2row = b2.reshape(1, D)

    p = _spmm(src, dst, feat, zeros_slab)
    h1 = _combine(p)
    p2 = _spmm(src, dst, h1, zeros_slab)
    return _combine_mlp(p2, W1, b1row, W2, b2row)
